# baseline (speedup 1.0000x reference)
import functools
import jax
import jax.numpy as jnp
from jax import lax
from jax.experimental import pallas as pl
from jax.experimental.pallas import tpu as pltpu
from jax.experimental.pallas import tpu_sc as plsc


def _build_sc_gather(B, C, T, V, D):
    info = plsc.get_sparse_core_info()
    NC, NS, L = info.num_cores, info.num_subcores, info.num_lanes
    NW = NC * NS
    N = B * C * T
    per_w = N // NW
    CH = 512
    n_chunks = per_w // CH

    mesh = plsc.VectorSubcoreMesh(core_axis_name="c", subcore_axis_name="s")

    @functools.partial(
        pl.kernel,
        mesh=mesh,
        compiler_params=pltpu.CompilerParams(use_tc_tiling_on_sc=False),
        out_type=jax.ShapeDtypeStruct((B, C, T, D), jnp.float32),
        scratch_types=[
            pltpu.VMEM((CH,), jnp.int32),
            pltpu.VMEM((CH,), jnp.int32),
            pltpu.VMEM((CH, D), jnp.float32),
            pltpu.VMEM((CH, D), jnp.float32),
            pltpu.SemaphoreType.DMA((2,)),
            pltpu.SemaphoreType.DMA((2,)),
        ],
    )
    def k(idx_hbm, table_hbm, out_hbm, idx_v0, idx_v1, rows_v0, rows_v1,
          gsem, osem):
        idx_v = (idx_v0, idx_v1)
        rows_v = (rows_v0, rows_v1)
        wid = lax.axis_index("s") * NC + lax.axis_index("c")

        def load_and_gather(ci, slot):
            g = wid * n_chunks + ci
            start = g * CH
            c = (start // T) % C
            off = (c * (V // C)).astype(jnp.int32)
            iv = idx_v[slot]
            pltpu.sync_copy(idx_hbm.at[pl.ds(start, CH)], iv)

            def add_body(j, _):
                sl = pl.ds(j * L, L)
                iv[sl] = iv[sl] + off
                return 0

            lax.fori_loop(0, CH // L, add_body, 0, unroll=True)
            pltpu.async_copy(
                table_hbm.at[idx_v[slot]], rows_v[slot], gsem.at[slot]
            )

        def out_slice(ci):
            start = (wid * n_chunks + ci) * CH
            b = start // (C * T)
            c = (start // T) % C
            t0 = start % T
            return out_hbm.at[b, c, pl.ds(t0, CH)]

        def gather_wait(slot):
            pltpu.make_async_copy(
                table_hbm.at[idx_v[slot]], rows_v[slot], gsem.at[slot]
            ).wait()

        def write_wait(slot):
            pltpu.make_async_copy(
                rows_v[slot], out_slice(0), osem.at[slot]
            ).wait()

        load_and_gather(0, 0)
        for g in range(n_chunks):
            slot = g % 2
            nslot = (g + 1) % 2
            if g + 1 < n_chunks:
                if g >= 1:
                    write_wait(nslot)
                load_and_gather(g + 1, nslot)
            gather_wait(slot)
            pltpu.async_copy(rows_v[slot], out_slice(g), osem.at[slot])
        write_wait(0)
        write_wait(1)

    return k


def kernel(input_ids, table):
    B_, C_, T_ = input_ids.shape
    V_, D_ = table.shape
    flat_idx = input_ids.reshape(B_ * C_ * T_).astype(jnp.int32)
    k = _build_sc_gather(B_, C_, T_, V_, D_)
    return k(flat_idx, table)


# speedup vs baseline: 1.0157x; 1.0157x over previous
import functools
import jax
import jax.numpy as jnp
from jax import lax
from jax.experimental import pallas as pl
from jax.experimental.pallas import tpu as pltpu
from jax.experimental.pallas import tpu_sc as plsc


def _build_sc_gather(B, C, T, V, D):
    info = plsc.get_sparse_core_info()
    NC, NS, L = info.num_cores, info.num_subcores, info.num_lanes
    NW = NC * NS
    N = B * C * T
    per_w = N // NW
    CH = 512
    n_chunks = per_w // CH
    NS = 3
    LA = NS - 1

    mesh = plsc.VectorSubcoreMesh(core_axis_name="c", subcore_axis_name="s")

    @functools.partial(
        pl.kernel,
        mesh=mesh,
        compiler_params=pltpu.CompilerParams(use_tc_tiling_on_sc=False),
        out_type=jax.ShapeDtypeStruct((B, C, T, D), jnp.float32),
        scratch_types=[
            pltpu.VMEM((CH,), jnp.int32),
            pltpu.VMEM((CH,), jnp.int32),
            pltpu.VMEM((CH,), jnp.int32),
            pltpu.VMEM((CH, D), jnp.float32),
            pltpu.VMEM((CH, D), jnp.float32),
            pltpu.VMEM((CH, D), jnp.float32),
            pltpu.SemaphoreType.DMA((NS,)),
            pltpu.SemaphoreType.DMA((NS,)),
        ],
    )
    def k(idx_hbm, table_hbm, out_hbm, idx_v0, idx_v1, idx_v2,
          rows_v0, rows_v1, rows_v2, gsem, osem):
        idx_v = (idx_v0, idx_v1, idx_v2)
        rows_v = (rows_v0, rows_v1, rows_v2)
        wid = lax.axis_index("s") * NC + lax.axis_index("c")

        def load_and_gather(ci, slot):
            g = wid * n_chunks + ci
            start = g * CH
            c = (start // T) % C
            off = (c * (V // C)).astype(jnp.int32)
            iv = idx_v[slot]
            pltpu.sync_copy(idx_hbm.at[pl.ds(start, CH)], iv)

            def add_body(j, _):
                sl = pl.ds(j * L, L)
                iv[sl] = iv[sl] + off
                return 0

            lax.fori_loop(0, CH // L, add_body, 0, unroll=True)
            pltpu.async_copy(
                table_hbm.at[idx_v[slot]], rows_v[slot], gsem.at[slot]
            )

        def out_slice(ci):
            start = (wid * n_chunks + ci) * CH
            b = start // (C * T)
            c = (start // T) % C
            t0 = start % T
            return out_hbm.at[b, c, pl.ds(t0, CH)]

        def gather_wait(slot):
            pltpu.make_async_copy(
                table_hbm.at[idx_v[slot]], rows_v[slot], gsem.at[slot]
            ).wait()

        def write_wait(ci, slot):
            pltpu.make_async_copy(
                rows_v[slot], out_slice(ci), osem.at[slot]
            ).wait()

        for g in range(n_chunks + LA):
            if g < n_chunks:
                s = g % NS
                if g >= NS:
                    write_wait(g - NS, s)
                load_and_gather(g, s)
            if g >= LA:
                gd = g - LA
                s = gd % NS
                gather_wait(s)
                pltpu.async_copy(rows_v[s], out_slice(gd), osem.at[s])
        for gd in range(n_chunks - NS, n_chunks):
            write_wait(gd, gd % NS)

    return k


def kernel(input_ids, table):
    B_, C_, T_ = input_ids.shape
    V_, D_ = table.shape
    flat_idx = input_ids.reshape(B_ * C_ * T_).astype(jnp.int32)
    k = _build_sc_gather(B_, C_, T_, V_, D_)
    return k(flat_idx, table)
